# baseline (speedup 1.0000x reference)
import functools
import jax
import jax.numpy as jnp
from jax import lax
from jax.experimental import pallas as pl
from jax.experimental.pallas import tpu as pltpu
from jax.experimental.pallas import tpu_sc as plsc

VOCAB = 1000000
DIM = 32
PADW = 128
B = 4096
F = 26
NROWS = B * F
NW = 32
RPW = NROWS // NW
LANES = 16
IDX_MINOR = 128
IDX_MAJOR = RPW // IDX_MINOR
CHUNK = 64
PAIRS = RPW // (2 * CHUNK)
EPS = 1e-12


def _rsqrt(v):
    i = lax.bitcast_convert_type(v, jnp.int32)
    i = jnp.int32(0x5F3759DF) - lax.shift_right_logical(i, 1)
    y = lax.bitcast_convert_type(i, jnp.float32)
    for _ in range(3):
        y = y * (1.5 - 0.5 * v * y * y)
    return y


def _allsum(v, perms):
    for p in perms:
        v = v + jnp.take_along_axis(v, p, axis=0, mode="promise_in_bounds")
    return v


def _make_kernel():
    mesh = plsc.VectorSubcoreMesh(core_axis_name="c", subcore_axis_name="s")

    @functools.partial(
        pl.kernel,
        mesh=mesh,
        out_type=jax.ShapeDtypeStruct((NROWS, DIM), jnp.float32),
        scratch_types=[
            pltpu.VMEM((IDX_MAJOR, IDX_MINOR), jnp.int32),
            pltpu.VMEM((CHUNK, PADW), jnp.float32),
            pltpu.VMEM((CHUNK, PADW), jnp.float32),
            pltpu.VMEM((CHUNK, DIM), jnp.float32),
            pltpu.VMEM((CHUNK, DIM), jnp.float32),
            pltpu.VMEM((DIM,), jnp.float32),
            pltpu.VMEM((DIM,), jnp.float32),
            pltpu.SemaphoreType.DMA,
            pltpu.SemaphoreType.DMA,
            pltpu.SemaphoreType.DMA,
            pltpu.SemaphoreType.DMA,
        ],
        compiler_params=pltpu.CompilerParams(
            needs_layout_passes=False, use_tc_tiling_on_sc=False),
    )
    def emb_ln(idx_hbm, table_hbm, gamma_hbm, beta_hbm, out_hbm,
               idx_v, blk0, blk1, out0, out1, gamma_v, beta_v,
               sem0, sem1, semo0, semo1):
        wid = lax.axis_index("s") * 2 + lax.axis_index("c")
        row_base = wid * RPW

        pltpu.sync_copy(idx_hbm.at[wid], idx_v)
        pltpu.sync_copy(gamma_hbm, gamma_v)
        pltpu.sync_copy(beta_hbm, beta_v)

        g_lo = gamma_v[pl.ds(0, LANES)]
        g_hi = gamma_v[pl.ds(LANES, LANES)]
        b_lo = beta_v[pl.ds(0, LANES)]
        b_hi = beta_v[pl.ds(LANES, LANES)]
        lane = lax.broadcasted_iota(jnp.int32, (LANES,), 0)
        perms = [lane ^ s for s in (8, 4, 2, 1)]

        def gather_chunk(p, half, blk, sem):
            pltpu.async_copy(
                table_hbm.at[idx_v.at[p, pl.ds(half * CHUNK, CHUNK)]],
                blk, sem)

        def drain(blk, sem):
            pltpu.make_async_copy(
                table_hbm.at[pl.ds(0, CHUNK)], blk, sem).wait()

        def drain_out(obuf, sem):
            pltpu.make_async_copy(
                obuf, out_hbm.at[pl.ds(0, CHUNK)], sem).wait()

        def compute(blk, obuf):
            for r in range(CHUNK):
                a = blk[r, pl.ds(0, LANES)]
                b = blk[r, pl.ds(LANES, LANES)]
                t = a + b
                u = a * a + b * b
                S = _allsum(t, perms)
                Q = _allsum(u, perms)
                mean = S * (1.0 / DIM)
                var = Q * (1.0 / DIM) - mean * mean
                pinv = _rsqrt(var + EPS)
                q = mean * pinv
                obuf[r, pl.ds(0, LANES)] = (a * pinv - q) * g_lo + b_lo
                obuf[r, pl.ds(LANES, LANES)] = (b * pinv - q) * g_hi + b_hi

        gather_chunk(0, 0, blk0, sem0)

        def body(p, carry):
            c0 = 2 * p
            gather_chunk(p, 1, blk1, sem1)
            drain(blk0, sem0)

            @pl.when(p > 0)
            def _():
                drain_out(out0, semo0)

            compute(blk0, out0)
            pltpu.async_copy(
                out0, out_hbm.at[pl.ds(row_base + c0 * CHUNK, CHUNK)], semo0)

            @pl.when(p + 1 < PAIRS)
            def _():
                gather_chunk(p + 1, 0, blk0, sem0)

            drain(blk1, sem1)

            @pl.when(p > 0)
            def _():
                drain_out(out1, semo1)

            compute(blk1, out1)
            pltpu.async_copy(
                out1, out_hbm.at[pl.ds(row_base + (c0 + 1) * CHUNK, CHUNK)],
                semo1)
            return carry

        lax.fori_loop(0, PAIRS, body, 0)
        drain_out(out0, semo0)
        drain_out(out1, semo1)

    return emb_ln


_EMB_LN = _make_kernel()


def kernel(input_ids, table, gamma, beta):
    tbl = jnp.pad(table, ((0, 0), (0, PADW - DIM)))
    idx = input_ids.astype(jnp.int32).reshape(NW, IDX_MAJOR, IDX_MINOR)
    out = _EMB_LN(idx, tbl, gamma, beta)
    return out.reshape(B, F, DIM)


# speedup vs baseline: 1.0290x; 1.0290x over previous
import functools
import jax
import jax.numpy as jnp
from jax import lax
from jax.experimental import pallas as pl
from jax.experimental.pallas import tpu as pltpu
from jax.experimental.pallas import tpu_sc as plsc

VOCAB = 1000000
DIM = 32
PADW = 128
B = 4096
F = 26
NROWS = B * F
NW = 32
RPW = NROWS // NW
BPW = RPW // F
LANES = 16
CHUNK = 104
CB = CHUNK // F
PAIRS = RPW // (2 * CHUNK)
EPS = 1e-12


def _rsqrt(v):
    i = lax.bitcast_convert_type(v, jnp.int32)
    i = jnp.int32(0x5F3759DF) - lax.shift_right_logical(i, 1)
    y = lax.bitcast_convert_type(i, jnp.float32)
    for _ in range(3):
        y = y * (1.5 - 0.5 * v * y * y)
    return y


def _splat_last(v):
    return jnp.take_along_axis(
        v, jnp.full((LANES,), LANES - 1, jnp.int32), axis=0,
        mode="promise_in_bounds")


def _make_kernel():
    mesh = plsc.VectorSubcoreMesh(core_axis_name="c", subcore_axis_name="s")

    @functools.partial(
        pl.kernel,
        mesh=mesh,
        out_type=jax.ShapeDtypeStruct((B, F, DIM), jnp.float32),
        scratch_types=[
            pltpu.VMEM((RPW,), jnp.int32),
            pltpu.VMEM((CHUNK, PADW), jnp.float32),
            pltpu.VMEM((CHUNK, PADW), jnp.float32),
            pltpu.VMEM((CB, F, DIM), jnp.float32),
            pltpu.VMEM((CB, F, DIM), jnp.float32),
            pltpu.VMEM((DIM,), jnp.float32),
            pltpu.VMEM((DIM,), jnp.float32),
            pltpu.SemaphoreType.DMA,
            pltpu.SemaphoreType.DMA,
            pltpu.SemaphoreType.DMA,
            pltpu.SemaphoreType.DMA,
        ],
        compiler_params=pltpu.CompilerParams(
            needs_layout_passes=False, use_tc_tiling_on_sc=False),
    )
    def emb_ln(idx_hbm, table_hbm, gamma_hbm, beta_hbm, out_hbm,
               idx_v, blk0, blk1, out0, out1, gamma_v, beta_v,
               sem0, sem1, semo0, semo1):
        wid = lax.axis_index("s") * 2 + lax.axis_index("c")
        batch_base = wid * BPW

        pltpu.sync_copy(idx_hbm.at[wid], idx_v)
        pltpu.sync_copy(gamma_hbm, gamma_v)
        pltpu.sync_copy(beta_hbm, beta_v)

        g_lo = gamma_v[pl.ds(0, LANES)]
        g_hi = gamma_v[pl.ds(LANES, LANES)]
        b_lo = beta_v[pl.ds(0, LANES)]
        b_hi = beta_v[pl.ds(LANES, LANES)]

        def gather_chunk(c, blk, sem):
            pltpu.async_copy(
                table_hbm.at[idx_v.at[pl.ds(c * CHUNK, CHUNK)]],
                blk, sem)

        def drain(blk, sem):
            pltpu.make_async_copy(
                table_hbm.at[pl.ds(0, CHUNK)], blk, sem).wait()

        def drain_out(obuf, sem):
            pltpu.make_async_copy(
                obuf, out_hbm.at[pl.ds(0, CB)], sem).wait()

        def compute(blk, obuf):
            for r in range(CHUNK):
                a = blk[r, pl.ds(0, LANES)]
                b = blk[r, pl.ds(LANES, LANES)]
                t = a + b
                u = a * a + b * b
                S = _splat_last(plsc.cumsum(t))
                Q = _splat_last(plsc.cumsum(u))
                mean = S * (1.0 / DIM)
                var = Q * (1.0 / DIM) - mean * mean
                pinv = _rsqrt(var + EPS)
                q = mean * pinv
                bb, ff = divmod(r, F)
                obuf[bb, ff, pl.ds(0, LANES)] = (a * pinv - q) * g_lo + b_lo
                obuf[bb, ff, pl.ds(LANES, LANES)] = (b * pinv - q) * g_hi + b_hi

        def put_out(c, obuf, sem):
            pltpu.async_copy(
                obuf, out_hbm.at[pl.ds(batch_base + c * CB, CB)], sem)

        gather_chunk(0, blk0, sem0)

        def body(p, carry):
            c0 = 2 * p
            gather_chunk(c0 + 1, blk1, sem1)
            drain(blk0, sem0)

            @pl.when(p > 0)
            def _():
                drain_out(out0, semo0)

            compute(blk0, out0)
            put_out(c0, out0, semo0)

            @pl.when(p + 1 < PAIRS)
            def _():
                gather_chunk(c0 + 2, blk0, sem0)

            drain(blk1, sem1)

            @pl.when(p > 0)
            def _():
                drain_out(out1, semo1)

            compute(blk1, out1)
            put_out(c0 + 1, out1, semo1)
            return carry

        lax.fori_loop(0, PAIRS, body, 0)
        drain_out(out0, semo0)
        drain_out(out1, semo1)

    return emb_ln


_EMB_LN = _make_kernel()


def kernel(input_ids, table, gamma, beta):
    tbl = jnp.pad(table, ((0, 0), (0, PADW - DIM)))
    idx = input_ids.astype(jnp.int32).reshape(NW, RPW)
    return _EMB_LN(idx, tbl, gamma, beta)


# speedup vs baseline: 1.0294x; 1.0004x over previous
import functools
import jax
import jax.numpy as jnp
from jax import lax
from jax.experimental import pallas as pl
from jax.experimental.pallas import tpu as pltpu
from jax.experimental.pallas import tpu_sc as plsc

VOCAB = 1000000
DIM = 32
PADW = 128
B = 4096
F = 26
NROWS = B * F
NW = 32
RPW = NROWS // NW
BPW = RPW // F
LANES = 16
CHUNK = 104
CB = CHUNK // F
PAIRS = RPW // (2 * CHUNK)
EPS = 1e-12


def _rsqrt(v):
    i = lax.bitcast_convert_type(v, jnp.int32)
    i = jnp.int32(0x5F3759DF) - lax.shift_right_logical(i, 1)
    y = lax.bitcast_convert_type(i, jnp.float32)
    for _ in range(3):
        y = y * (1.5 - 0.5 * v * y * y)
    return y


def _splat_last(v):
    return jnp.take_along_axis(
        v, jnp.full((LANES,), LANES - 1, jnp.int32), axis=0,
        mode="promise_in_bounds")


def _make_kernel():
    mesh = plsc.VectorSubcoreMesh(core_axis_name="c", subcore_axis_name="s")

    @functools.partial(
        pl.kernel,
        mesh=mesh,
        out_type=jax.ShapeDtypeStruct((B, F, DIM), jnp.float32),
        scratch_types=[
            pltpu.VMEM((RPW,), jnp.int32),
            pltpu.VMEM((CHUNK, PADW), jnp.float32),
            pltpu.VMEM((CHUNK, PADW), jnp.float32),
            pltpu.VMEM((CB, F, DIM), jnp.float32),
            pltpu.VMEM((CB, F, DIM), jnp.float32),
            pltpu.VMEM((DIM,), jnp.float32),
            pltpu.VMEM((DIM,), jnp.float32),
            pltpu.SemaphoreType.DMA,
            pltpu.SemaphoreType.DMA,
            pltpu.SemaphoreType.DMA,
            pltpu.SemaphoreType.DMA,
        ],
        compiler_params=pltpu.CompilerParams(
            needs_layout_passes=False, use_tc_tiling_on_sc=False),
    )
    def emb_ln(idx_hbm, table_hbm, gamma_hbm, beta_hbm, out_hbm,
               idx_v, blk0, blk1, out0, out1, gamma_v, beta_v,
               sem0, sem1, semo0, semo1):
        wid = lax.axis_index("s") * 2 + lax.axis_index("c")
        batch_base = wid * BPW

        pltpu.sync_copy(idx_hbm.at[wid], idx_v)
        pltpu.sync_copy(gamma_hbm, gamma_v)
        pltpu.sync_copy(beta_hbm, beta_v)

        g_lo = gamma_v[pl.ds(0, LANES)]
        g_hi = gamma_v[pl.ds(LANES, LANES)]
        b_lo = beta_v[pl.ds(0, LANES)]
        b_hi = beta_v[pl.ds(LANES, LANES)]

        def gather_chunk(c, blk, sem):
            pltpu.async_copy(
                table_hbm.at[idx_v.at[pl.ds(c * CHUNK, CHUNK)]],
                blk, sem)

        def drain(blk, sem):
            pltpu.make_async_copy(
                table_hbm.at[pl.ds(0, CHUNK)], blk, sem).wait()

        def drain_out(obuf, sem):
            pltpu.make_async_copy(
                obuf, out_hbm.at[pl.ds(0, CB)], sem).wait()

        GRP = 8

        def compute(blk, obuf):
            for g0 in range(0, CHUNK, GRP):
                rows = range(g0, g0 + GRP)
                asl = [blk[r, pl.ds(0, LANES)] for r in rows]
                bsl = [blk[r, pl.ds(LANES, LANES)] for r in rows]
                ys = []
                for i, r in enumerate(rows):
                    a, b = asl[i], bsl[i]
                    t = a + b
                    u = a * a + b * b
                    S = _splat_last(plsc.cumsum(t))
                    Q = _splat_last(plsc.cumsum(u))
                    mean = S * (1.0 / DIM)
                    var = Q * (1.0 / DIM) - mean * mean
                    pinv = _rsqrt(var + EPS)
                    q = mean * pinv
                    ys.append(((a * pinv - q) * g_lo + b_lo,
                               (b * pinv - q) * g_hi + b_hi))
                for i, r in enumerate(rows):
                    bb, ff = divmod(r, F)
                    obuf[bb, ff, pl.ds(0, LANES)] = ys[i][0]
                    obuf[bb, ff, pl.ds(LANES, LANES)] = ys[i][1]

        def put_out(c, obuf, sem):
            pltpu.async_copy(
                obuf, out_hbm.at[pl.ds(batch_base + c * CB, CB)], sem)

        gather_chunk(0, blk0, sem0)

        def body(p, carry):
            c0 = 2 * p
            gather_chunk(c0 + 1, blk1, sem1)
            drain(blk0, sem0)

            @pl.when(p > 0)
            def _():
                drain_out(out0, semo0)

            compute(blk0, out0)
            put_out(c0, out0, semo0)

            @pl.when(p + 1 < PAIRS)
            def _():
                gather_chunk(c0 + 2, blk0, sem0)

            drain(blk1, sem1)

            @pl.when(p > 0)
            def _():
                drain_out(out1, semo1)

            compute(blk1, out1)
            put_out(c0 + 1, out1, semo1)
            return carry

        lax.fori_loop(0, PAIRS, body, 0)
        drain_out(out0, semo0)
        drain_out(out1, semo1)

    return emb_ln


_EMB_LN = _make_kernel()


def kernel(input_ids, table, gamma, beta):
    tbl = jnp.pad(table, ((0, 0), (0, PADW - DIM)))
    idx = input_ids.astype(jnp.int32).reshape(NW, RPW)
    return _EMB_LN(idx, tbl, gamma, beta)
